# baseline (speedup 1.0000x reference)
import jax
import jax.numpy as jnp
from jax import lax
from jax.experimental import pallas as pl

_BS, _NR, _NX = 8, 64, 32
_NN = _NR * _NX
_CR = 8
_CN = _CR * _NX
_GU = _NR // _CR
_NEG = -1e30

_F32 = jnp.float32


def _mm(a, b):
    return jnp.dot(a, b, preferred_element_type=_F32)


def _mm_t(w, x):
    return lax.dot_general(w, x, (((0,), (1,)), ((), ())),
                           preferred_element_type=_F32)


def _softmax_rows(s):
    m = jnp.max(s, axis=-1, keepdims=True)
    e = jnp.exp(s - m)
    return e / (jnp.sum(e, axis=-1, keepdims=True) + 1e-16)


def _attend(h, a_st, a_d, out, maskbias):
    s_parts = []
    for u in range(_GU):
        for g in range(2):
            row = a_st[g:g + 1, u * _CN:(u + 1) * _CN]
            col = a_d[u * _CN:(u + 1) * _CN, g:g + 1]
            s_parts.append(row + col)
    s = jnp.concatenate(s_parts, axis=0)
    s = jnp.maximum(s, 0.2 * s) + maskbias
    coef = _softmax_rows(s)
    x_parts = []
    for u in range(_GU):
        aggs = [_mm(coef[(2 * u + g) * _CN:(2 * u + g + 1) * _CN, :],
                    h[u * _CN:(u + 1) * _CN, g * out:(g + 1) * out])
                for g in range(2)]
        x_parts.append(jnp.concatenate(aggs, axis=-1))
    return jnp.concatenate(x_parts, axis=0)


def _body(pairs_ref, xs_ref,
          f1w, f1b, f2w, f2b, f3w, f3b, f4w, f4b, f5w, f5b,
          g1w, g1b, g2w, g2b,
          h1w, h1b, h2w, h2b, h3w, h3b, h4w, h4b, h5w, h5b,
          wa1w, wa1b, wb1w, wb1b,
          wl0w, wl0b, wbias0w, wbias0b,
          wsrc0, bsrc0, wdst0, bdst0,
          w2blin, gsum, b2w1t,
          wsrc1, bsrc1, wdst1, bdst1,
          wbias1w, wbias1b, wo, bo,
          out_ref):
    relu = lambda v: jnp.maximum(v, 0.0)

    x = pairs_ref[...]
    x = relu(_mm(x, f1w[...]) + f1b[...])
    x = x + relu(_mm(x, f2w[...]) + f2b[...])
    x = x + relu(_mm(x, f3w[...]) + f3b[...])
    x = x + relu(_mm(x, f4w[...]) + f4b[...])
    x = relu(_mm(x, f5w[...]) + f5b[...])
    x = jnp.mean(x.reshape(_BS * _NR, _NX, 64), axis=1)
    x = relu(_mm(x, g1w[...]) + g1b[...])
    x = relu(_mm(x, g2w[...]) + g2b[...])
    x = jnp.mean(x.reshape(_BS, _NR, 64), axis=1)
    x = relu(_mm(x, h1w[...]) + h1b[...])
    x = x + relu(_mm(x, h2w[...]) + h2b[...])
    x = x + relu(_mm(x, h3w[...]) + h3b[...])
    x = x + relu(_mm(x, h4w[...]) + h4b[...])
    d2v = relu(_mm(x, h5w[...]) + h5b[...])

    inner0 = relu(_mm(d2v, wa1w[...]) + wa1b[...])
    inner1 = relu(_mm(d2v, wb1w[...]) + wb1b[...])

    w0row = _mm(inner0, wl0w[...]) + wl0b[...]
    bias0 = _mm(inner0, wbias0w[...]) + wbias0b[...]
    src0c = _mm_t(wsrc0[...], inner0) + bsrc0[...]
    dst0c = _mm_t(wdst0[...], inner0) + bdst0[...]

    lhs1 = jnp.broadcast_to(inner1[:, None, :], (_BS, 64, 64))
    lhs1 = lhs1.reshape(_BS * 64, 64)
    kk = lax.broadcasted_iota(jnp.int32, (_BS * 64, 1024), 0) % 64
    kp = lax.broadcasted_iota(jnp.int32, (_BS * 64, 1024), 1) % 64
    f_all = jnp.where(kk == kp, _mm(lhs1, w2blin[...]), 0.0)
    b2t = jnp.broadcast_to(b2w1t[...][None], (_BS, 64, 16)).reshape(_BS * 64, 16)
    w1t_all = _mm(f_all, gsum[...]) + b2t

    src1c = _mm_t(wsrc1[...], inner1) + bsrc1[...]
    dst1c = _mm_t(wdst1[...], inner1) + bdst1[...]
    bias1 = _mm(inner1, wbias1w[...]) + wbias1b[...]

    nw = _GU * 2 * _CN
    ii = (lax.broadcasted_iota(jnp.int32, (nw, _CN), 0) % _CN) // _NX
    jj = lax.broadcasted_iota(jnp.int32, (nw, _CN), 1) // _NX
    maskbias = jnp.where(ii == jj, 0.0, _NEG)

    for b in range(_BS):
        xcol = xs_ref[b * _NN:(b + 1) * _NN]
        h = xcol * w0row[b:b + 1, :]
        asrc = jnp.concatenate(
            [src0c[0:64, b:b + 1], src0c[64:128, b:b + 1]], axis=1)
        adst = jnp.concatenate(
            [dst0c[0:64, b:b + 1], dst0c[64:128, b:b + 1]], axis=1)
        a_st = _mm_t(asrc, h)
        a_d = _mm(h, adst)
        x2 = _attend(h, a_st, a_d, 32, maskbias) + bias0[b:b + 1, :]

        h2 = _mm(x2, w1t_all[b * 64:(b + 1) * 64, :])
        asrc1 = jnp.concatenate(
            [src1c[0:16, b:b + 1], src1c[16:32, b:b + 1]], axis=1)
        adst1 = jnp.concatenate(
            [dst1c[0:16, b:b + 1], dst1c[16:32, b:b + 1]], axis=1)
        a_s2t = _mm_t(asrc1, h2)
        a_d2 = _mm(h2, adst1)
        x3 = _attend(h2, a_s2t, a_d2, 8, maskbias) + bias1[b:b + 1, :]

        row = jnp.sum(x3.reshape(_NR, _NX, 16), axis=1)
        out_ref[b * _NR:(b + 1) * _NR, :] = _mm(row, wo[...]) + bo[...]


@jax.jit
def kernel(xs, pairs, params):
    p = params

    def wt(name):
        W, b = p[name]
        return W.T.astype(_F32), b.reshape(1, -1).astype(_F32)

    names = ["f1", "f2r", "f3r", "f4r", "f5", "g1", "g2",
             "h1", "h2r", "h3r", "h4r", "h5", "wg0_1", "wg1_1"]
    wargs = []
    for n in names:
        W, b = wt(n)
        wargs += [W, b]

    W2a, b2a = p["wg0_2"]
    W2a = W2a.T.astype(_F32)
    b2a = b2a.astype(_F32)
    W2b, b2b = p["wg1_2"]
    W2b = W2b.T.astype(_F32)
    b2b = b2b.astype(_F32)

    wl0w = W2a[:, 0:128:2]
    wl0b = b2a[0:128:2].reshape(1, 64)
    wbias0w = W2a[:, 256:320]
    wbias0b = b2a[256:320].reshape(1, 64)

    def padded_cols(W2, b2, base, heads, out):
        n = heads * out
        wcols, bcols = [], []
        for g in range(2):
            wpad = jnp.zeros((64, n), _F32)
            wpad = wpad.at[:, g * out:(g + 1) * out].set(
                W2[:, base + g * out: base + (g + 1) * out])
            bpad = jnp.zeros((n,), _F32)
            bpad = bpad.at[g * out:(g + 1) * out].set(
                b2[base + g * out: base + (g + 1) * out])
            wcols.append(wpad)
            bcols.append(bpad)
        wfull = jnp.concatenate(wcols, axis=1)
        bfull = jnp.concatenate(bcols).reshape(2 * n, 1)
        return wfull, bfull

    wsrc0, bsrc0 = padded_cols(W2a, b2a, 128, 2, 32)
    wdst0, bdst0 = padded_cols(W2a, b2a, 192, 2, 32)
    wsrc1, bsrc1 = padded_cols(W2b, b2b, 1024, 2, 8)
    wdst1, bdst1 = padded_cols(W2b, b2b, 1040, 2, 8)

    w2blin = W2b[:, :1024]
    gsum = jnp.repeat(jnp.eye(16, dtype=_F32), 64, axis=0)
    b2w1t = b2b[:1024].reshape(16, 64).T

    wbias1w = W2b[:, 1056:1072]
    wbias1b = b2b[1056:1072].reshape(1, 16)

    Wo, bo = p["out_lin"]
    WoT = Wo.T.astype(_F32)
    bo = bo.reshape(1, 2).astype(_F32)

    pairs_f = pairs.reshape(_BS * _NN, 2)
    xs_f = xs.reshape(_BS * _NN, 1).astype(_F32)

    out = pl.pallas_call(
        _body,
        out_shape=jax.ShapeDtypeStruct((_BS * _NR, 2), _F32),
    )(pairs_f, xs_f, *wargs,
      wl0w, wl0b, wbias0w, wbias0b,
      wsrc0, bsrc0, wdst0, bdst0,
      w2blin, gsum, b2w1t,
      wsrc1, bsrc1, wdst1, bdst1,
      wbias1w, wbias1b, WoT, bo)
    return out.reshape(_BS, _NR, 2)


# speedup vs baseline: 1.2591x; 1.2591x over previous
import jax
import jax.numpy as jnp
from jax import lax
from jax.experimental import pallas as pl
from jax.experimental.pallas import tpu as pltpu

_BS, _NR, _NX = 8, 64, 32
_NN = _NR * _NX
_CR = 8
_CN = _CR * _NX
_GU = _NR // _CR
_NEG = -1e30

_F32 = jnp.float32


def _mm(a, b):
    return jnp.dot(a, b, preferred_element_type=_F32)


def _mm_t(w, x):
    return lax.dot_general(w, x, (((0,), (1,)), ((), ())),
                           preferred_element_type=_F32)


def _d2v_body(pairs_ref,
              f1w, f1b, f2w, f2b, f3w, f3b, f4w, f4b, f5w, f5b,
              g1w, g1b, g2w, g2b,
              h1w, h1b, h2w, h2b, h3w, h3b, h4w, h4b, h5w, h5b,
              wa1w, wa1b, wa2w, wa2b, wb1w, wb1b, wb2w, wb2b,
              w0_out, w1_out):
    relu = lambda v: jnp.maximum(v, 0.0)
    x = pairs_ref[...]
    x = relu(_mm(x, f1w[...]) + f1b[...])
    x = x + relu(_mm(x, f2w[...]) + f2b[...])
    x = x + relu(_mm(x, f3w[...]) + f3b[...])
    x = x + relu(_mm(x, f4w[...]) + f4b[...])
    x = relu(_mm(x, f5w[...]) + f5b[...])
    x = jnp.mean(x.reshape(_BS * _NR, _NX, 64), axis=1)
    x = relu(_mm(x, g1w[...]) + g1b[...])
    x = relu(_mm(x, g2w[...]) + g2b[...])
    x = jnp.mean(x.reshape(_BS, _NR, 64), axis=1)
    x = relu(_mm(x, h1w[...]) + h1b[...])
    x = x + relu(_mm(x, h2w[...]) + h2b[...])
    x = x + relu(_mm(x, h3w[...]) + h3b[...])
    x = x + relu(_mm(x, h4w[...]) + h4b[...])
    d2v = relu(_mm(x, h5w[...]) + h5b[...])
    w0_out[...] = _mm(relu(_mm(d2v, wa1w[...]) + wa1b[...]),
                      wa2w[...]) + wa2b[...]
    w1_out[...] = _mm(relu(_mm(d2v, wb1w[...]) + wb1b[...]),
                      wb2w[...]) + wb2b[...]


def _attend(h, a_st, a_d, a_s, out, maskbias):
    m_blk = jnp.max(a_s.reshape(_NR, _NX, 2), axis=1)
    s_parts, ad_parts, ms_parts = [], [], []
    for u in range(_GU):
        for g in range(2):
            row = a_st[g:g + 1, u * _CN:(u + 1) * _CN]
            col = a_d[u * _CN:(u + 1) * _CN, g:g + 1]
            s_parts.append(row + col)
            ad_parts.append(col)
            mb = m_blk[u * _CR:(u + 1) * _CR, g:g + 1]
            ms_parts.append(jnp.broadcast_to(
                mb[:, None, :], (_CR, _NX, 1)).reshape(_CN, 1))
    s = jnp.concatenate(s_parts, axis=0)
    mcol = jnp.concatenate(ms_parts, axis=0) + jnp.concatenate(ad_parts, axis=0)
    mcol = jnp.maximum(mcol, 0.2 * mcol)
    s = jnp.maximum(s, 0.2 * s)
    e = jnp.exp(s - mcol + maskbias)
    den = _mm(e, jnp.ones((_CN, 1), _F32))
    r = 1.0 / (den + 1e-16)
    x_parts = []
    for u in range(_GU):
        aggs = []
        for g in range(2):
            k = 2 * u + g
            agg = _mm(e[k * _CN:(k + 1) * _CN, :],
                      h[u * _CN:(u + 1) * _CN, g * out:(g + 1) * out])
            aggs.append(agg * r[k * _CN:(k + 1) * _CN, :])
        x_parts.append(jnp.concatenate(aggs, axis=-1))
    return jnp.concatenate(x_parts, axis=0)


def _gnn_body(xcol_ref, w0r_ref,
              as0_ref, ad0_ref, b0_ref,
              w1t_ref, as1_ref, ad1_ref, b1_ref,
              wo_ref, bo_ref, out_ref, mb_ref):
    nw = _GU * 2 * _CN
    b = pl.program_id(0)

    @pl.when(b == 0)
    def _init_mask():
        ii = (lax.broadcasted_iota(jnp.int32, (nw, _CN), 0) % _CN) // _NX
        jj = lax.broadcasted_iota(jnp.int32, (nw, _CN), 1) // _NX
        mb_ref[...] = jnp.where(ii == jj, 0.0, _NEG)

    maskbias = mb_ref[...]

    xcol = xcol_ref[0]
    h = xcol * w0r_ref[0]
    a_st = _mm_t(as0_ref[0], h)
    a_s = _mm(h, as0_ref[0])
    a_d = _mm(h, ad0_ref[0])
    x2 = _attend(h, a_st, a_d, a_s, 32, maskbias) + b0_ref[0]

    h2 = _mm(x2, w1t_ref[0])
    a_s2t = _mm_t(as1_ref[0], h2)
    a_s2 = _mm(h2, as1_ref[0])
    a_d2 = _mm(h2, ad1_ref[0])
    x3 = _attend(h2, a_s2t, a_d2, a_s2, 8, maskbias) + b1_ref[0]

    row = jnp.sum(x3.reshape(_NR, _NX, 16), axis=1)
    out_ref[0] = _mm(row, wo_ref[...]) + bo_ref[...]


def _full(shape):
    nd = len(shape)
    return pl.BlockSpec(shape, lambda b: (0,) * nd)


def _per_batch(shape):
    nd = len(shape)
    return pl.BlockSpec((1,) + shape, lambda b: (b,) + (0,) * nd)


@jax.jit
def kernel(xs, pairs, params):
    p = params

    def wt(name):
        W, b = p[name]
        return W.T.astype(_F32), b.reshape(1, -1).astype(_F32)

    names = ["f1", "f2r", "f3r", "f4r", "f5", "g1", "g2",
             "h1", "h2r", "h3r", "h4r", "h5",
             "wg0_1", "wg0_2", "wg1_1", "wg1_2"]
    wargs = []
    for n in names:
        W, b = wt(n)
        wargs += [W, b]

    pairs_f = pairs.reshape(_BS * _NN, 2)
    w0, w1 = pl.pallas_call(
        _d2v_body,
        out_shape=[jax.ShapeDtypeStruct((_BS, 320), _F32),
                   jax.ShapeDtypeStruct((_BS, 1072), _F32)],
    )(pairs_f, *wargs)

    lin_w0 = w0[:, :128].reshape(_BS, 64, 2)
    w0row = lin_w0[:, :, 0].reshape(_BS, 1, 64)
    a_src0 = w0[:, 128:192].reshape(_BS, 2, 32)
    a_dst0 = w0[:, 192:256].reshape(_BS, 2, 32)
    bias0 = w0[:, 256:320].reshape(_BS, 1, 64)

    z32 = jnp.zeros((_BS, 32), _F32)
    A_src0 = jnp.stack(
        [jnp.concatenate([a_src0[:, 0, :], z32], axis=1),
         jnp.concatenate([z32, a_src0[:, 1, :]], axis=1)], axis=-1)
    A_dst0 = jnp.stack(
        [jnp.concatenate([a_dst0[:, 0, :], z32], axis=1),
         jnp.concatenate([z32, a_dst0[:, 1, :]], axis=1)], axis=-1)

    lin_w1 = w1[:, :1024].reshape(_BS, 16, 64)
    W1T = jnp.transpose(lin_w1, (0, 2, 1))
    a_src1 = w1[:, 1024:1040].reshape(_BS, 2, 8)
    a_dst1 = w1[:, 1040:1056].reshape(_BS, 2, 8)
    bias1 = w1[:, 1056:1072].reshape(_BS, 1, 16)

    z8 = jnp.zeros((_BS, 8), _F32)
    A_src1 = jnp.stack(
        [jnp.concatenate([a_src1[:, 0, :], z8], axis=1),
         jnp.concatenate([z8, a_src1[:, 1, :]], axis=1)], axis=-1)
    A_dst1 = jnp.stack(
        [jnp.concatenate([a_dst1[:, 0, :], z8], axis=1),
         jnp.concatenate([z8, a_dst1[:, 1, :]], axis=1)], axis=-1)

    Wo, bo = p["out_lin"]
    WoT = Wo.T.astype(_F32)
    bo = bo.reshape(1, 2).astype(_F32)

    xcol = xs.reshape(_BS, _NN, 1).astype(_F32)

    out = pl.pallas_call(
        _gnn_body,
        grid=(_BS,),
        in_specs=[_per_batch((_NN, 1)),
                  _per_batch((1, 64)),
                  _per_batch((64, 2)), _per_batch((64, 2)),
                  _per_batch((1, 64)),
                  _per_batch((64, 16)),
                  _per_batch((16, 2)), _per_batch((16, 2)),
                  _per_batch((1, 16)),
                  _full((16, 2)), _full((1, 2))],
        out_specs=_per_batch((_NR, 2)),
        out_shape=jax.ShapeDtypeStruct((_BS, _NR, 2), _F32),
        scratch_shapes=[pltpu.VMEM((_GU * 2 * _CN, _CN), _F32)],
    )(xcol, w0row,
      A_src0, A_dst0, bias0,
      W1T, A_src1, A_dst1, bias1, WoT, bo)
    return out


# speedup vs baseline: 1.3499x; 1.0721x over previous
import jax
import jax.numpy as jnp
from jax import lax
from jax.experimental import pallas as pl

_BS, _NR, _NX = 8, 64, 32
_NN = _NR * _NX
_CR = 8
_CN = _CR * _NX
_GU = _NR // _CR
_NEG = -1e30

_F32 = jnp.float32


def _mm(a, b):
    return jnp.dot(a, b, preferred_element_type=_F32)


def _mm_t(w, x):
    return lax.dot_general(w, x, (((0,), (1,)), ((), ())),
                           preferred_element_type=_F32)


def _mm_rt(a, w):
    return lax.dot_general(a, w, (((1,), (1,)), ((), ())),
                           preferred_element_type=_F32)


def _d2v_body(pairs_ref,
              f1w, f1b, f2w, f2b, f3w, f3b, f4w, f4b, f5w, f5b,
              g1w, g1b, g2w, g2b,
              h1w, h1b, h2w, h2b, h3w, h3b, h4w, h4b, h5w, h5b,
              wa1w, wa1b, wa2w, wa2b, wb1w, wb1b, wb2w, wb2b,
              w0_out, w1_out):
    relu = lambda v: jnp.maximum(v, 0.0)
    x = pairs_ref[...]
    x = relu(_mm_rt(x, f1w[...]) + f1b[...])
    x = x + relu(_mm_rt(x, f2w[...]) + f2b[...])
    x = x + relu(_mm_rt(x, f3w[...]) + f3b[...])
    x = x + relu(_mm_rt(x, f4w[...]) + f4b[...])
    x = relu(_mm_rt(x, f5w[...]) + f5b[...])
    x = jnp.mean(x.reshape(_BS * _NR, _NX, 64), axis=1)
    x = relu(_mm_rt(x, g1w[...]) + g1b[...])
    x = relu(_mm_rt(x, g2w[...]) + g2b[...])
    x = jnp.mean(x.reshape(_BS, _NR, 64), axis=1)
    x = relu(_mm_rt(x, h1w[...]) + h1b[...])
    x = x + relu(_mm_rt(x, h2w[...]) + h2b[...])
    x = x + relu(_mm_rt(x, h3w[...]) + h3b[...])
    x = x + relu(_mm_rt(x, h4w[...]) + h4b[...])
    d2v = relu(_mm_rt(x, h5w[...]) + h5b[...])
    w0_out[...] = _mm_rt(relu(_mm_rt(d2v, wa1w[...]) + wa1b[...]),
                         wa2w[...]) + wa2b[...]
    w1_out[...] = _mm_rt(relu(_mm_rt(d2v, wb1w[...]) + wb1b[...]),
                         wb2w[...]) + wb2b[...]


def _softmax_rows(s):
    m = jnp.max(s, axis=-1, keepdims=True)
    e = jnp.exp(s - m)
    return e / (jnp.sum(e, axis=-1, keepdims=True) + 1e-16)


def _attend(h, a_st, a_d, out, masked):
    s_parts = []
    for u in range(_GU):
        for g in range(2):
            row = a_st[g:g + 1, u * _CN:(u + 1) * _CN]
            col = a_d[u * _CN:(u + 1) * _CN, g:g + 1]
            s_parts.append(row + col)
    coef = _softmax_rows(masked(jnp.concatenate(s_parts, axis=0)))
    x_parts = []
    for u in range(_GU):
        aggs = [_mm(coef[(2 * u + g) * _CN:(2 * u + g + 1) * _CN, :],
                    h[u * _CN:(u + 1) * _CN, g * out:(g + 1) * out])
                for g in range(2)]
        x_parts.append(jnp.concatenate(aggs, axis=-1))
    return jnp.concatenate(x_parts, axis=0)


def _gnn_body(xcol_ref, w0r_ref,
              as0_ref, ad0_ref, b0_ref,
              w1t_ref, as1_ref, ad1_ref, b1_ref,
              wo_ref, bo_ref, out_ref):
    nw = _GU * 2 * _CN
    ii = (lax.broadcasted_iota(jnp.int32, (nw, _CN), 0) % _CN) // _NX
    jj = lax.broadcasted_iota(jnp.int32, (nw, _CN), 1) // _NX
    mask = ii == jj

    def masked(s):
        s = jnp.where(s >= 0.0, s, 0.2 * s)
        return jnp.where(mask, s, _NEG)

    xcol = xcol_ref[0]
    h = xcol * w0r_ref[0]
    a_st = _mm_t(as0_ref[0], h)
    a_d = _mm(h, ad0_ref[0])
    x2 = _attend(h, a_st, a_d, 32, masked) + b0_ref[0]

    h2 = _mm(x2, w1t_ref[0])
    a_s2t = _mm_t(as1_ref[0], h2)
    a_d2 = _mm(h2, ad1_ref[0])
    x3 = _attend(h2, a_s2t, a_d2, 8, masked) + b1_ref[0]

    row = jnp.sum(x3.reshape(_NR, _NX, 16), axis=1)
    out_ref[0] = _mm(row, wo_ref[...]) + bo_ref[...]


def _full(shape):
    nd = len(shape)
    return pl.BlockSpec(shape, lambda b: (0,) * nd)


def _per_batch(shape):
    nd = len(shape)
    return pl.BlockSpec((1,) + shape, lambda b: (b,) + (0,) * nd)


@jax.jit
def kernel(xs, pairs, params):
    p = params

    def wt(name):
        W, b = p[name]
        return W.astype(_F32), b.reshape(1, -1).astype(_F32)

    names = ["f1", "f2r", "f3r", "f4r", "f5", "g1", "g2",
             "h1", "h2r", "h3r", "h4r", "h5",
             "wg0_1", "wg0_2", "wg1_1", "wg1_2"]
    wargs = []
    for n in names:
        W, b = wt(n)
        wargs += [W, b]

    pairs_f = pairs.reshape(_BS * _NN, 2)
    w0, w1 = pl.pallas_call(
        _d2v_body,
        out_shape=[jax.ShapeDtypeStruct((_BS, 320), _F32),
                   jax.ShapeDtypeStruct((_BS, 1072), _F32)],
    )(pairs_f, *wargs)

    lin_w0 = w0[:, :128].reshape(_BS, 64, 2)
    w0row = lin_w0[:, :, 0].reshape(_BS, 1, 64)
    a_src0 = w0[:, 128:192].reshape(_BS, 2, 32)
    a_dst0 = w0[:, 192:256].reshape(_BS, 2, 32)
    bias0 = w0[:, 256:320].reshape(_BS, 1, 64)

    z32 = jnp.zeros((_BS, 32), _F32)
    A_src0 = jnp.stack(
        [jnp.concatenate([a_src0[:, 0, :], z32], axis=1),
         jnp.concatenate([z32, a_src0[:, 1, :]], axis=1)], axis=-1)
    A_dst0 = jnp.stack(
        [jnp.concatenate([a_dst0[:, 0, :], z32], axis=1),
         jnp.concatenate([z32, a_dst0[:, 1, :]], axis=1)], axis=-1)

    lin_w1 = w1[:, :1024].reshape(_BS, 16, 64)
    W1T = jnp.transpose(lin_w1, (0, 2, 1))
    a_src1 = w1[:, 1024:1040].reshape(_BS, 2, 8)
    a_dst1 = w1[:, 1040:1056].reshape(_BS, 2, 8)
    bias1 = w1[:, 1056:1072].reshape(_BS, 1, 16)

    z8 = jnp.zeros((_BS, 8), _F32)
    A_src1 = jnp.stack(
        [jnp.concatenate([a_src1[:, 0, :], z8], axis=1),
         jnp.concatenate([z8, a_src1[:, 1, :]], axis=1)], axis=-1)
    A_dst1 = jnp.stack(
        [jnp.concatenate([a_dst1[:, 0, :], z8], axis=1),
         jnp.concatenate([z8, a_dst1[:, 1, :]], axis=1)], axis=-1)

    Wo, bo = p["out_lin"]
    WoT = Wo.T.astype(_F32)
    bo = bo.reshape(1, 2).astype(_F32)

    xcol = xs.reshape(_BS, _NN, 1).astype(_F32)

    out = pl.pallas_call(
        _gnn_body,
        grid=(_BS,),
        in_specs=[_per_batch((_NN, 1)),
                  _per_batch((1, 64)),
                  _per_batch((64, 2)), _per_batch((64, 2)),
                  _per_batch((1, 64)),
                  _per_batch((64, 16)),
                  _per_batch((16, 2)), _per_batch((16, 2)),
                  _per_batch((1, 16)),
                  _full((16, 2)), _full((1, 2))],
        out_specs=_per_batch((_NR, 2)),
        out_shape=jax.ShapeDtypeStruct((_BS, _NR, 2), _F32),
    )(xcol, w0row,
      A_src0, A_dst0, bias0,
      W1T, A_src1, A_dst1, bias1, WoT, bo)
    return out


# speedup vs baseline: 1.6391x; 1.2143x over previous
import jax
import jax.numpy as jnp
from jax import lax
from jax.experimental import pallas as pl

_BS, _NR, _NX = 8, 64, 32
_NN = _NR * _NX
_CR = 8
_CN = _CR * _NX
_GU = _NR // _CR
_NEG = -1e30

_F32 = jnp.float32


def _mm(a, b):
    return jnp.dot(a, b, preferred_element_type=_F32)


def _mm_t(w, x):
    return lax.dot_general(w, x, (((0,), (1,)), ((), ())),
                           preferred_element_type=_F32)


def _mm_rt(a, w):
    return lax.dot_general(a, w, (((1,), (1,)), ((), ())),
                           preferred_element_type=_F32)


def _d2v_body(pairs_ref,
              f1w, f1b, f2w, f2b, f3w, f3b, f4w, f4b, f5w, f5b,
              g1w, g1b, g2w, g2b,
              h1w, h1b, h2w, h2b, h3w, h3b, h4w, h4b, h5w, h5b,
              wa1w, wa1b, wa2w, wa2b, wb1w, wb1b, wb2w, wb2b,
              w0_out, w1_out):
    relu = lambda v: jnp.maximum(v, 0.0)
    x = pairs_ref[...]
    x = relu(_mm_rt(x, f1w[...]) + f1b[...])
    x = x + relu(_mm_rt(x, f2w[...]) + f2b[...])
    x = x + relu(_mm_rt(x, f3w[...]) + f3b[...])
    x = x + relu(_mm_rt(x, f4w[...]) + f4b[...])
    x = relu(_mm_rt(x, f5w[...]) + f5b[...])
    x = jnp.mean(x.reshape(_BS * _NR, _NX, 64), axis=1)
    x = relu(_mm_rt(x, g1w[...]) + g1b[...])
    x = relu(_mm_rt(x, g2w[...]) + g2b[...])
    x = jnp.mean(x.reshape(_BS, _NR, 64), axis=1)
    x = relu(_mm_rt(x, h1w[...]) + h1b[...])
    x = x + relu(_mm_rt(x, h2w[...]) + h2b[...])
    x = x + relu(_mm_rt(x, h3w[...]) + h3b[...])
    x = x + relu(_mm_rt(x, h4w[...]) + h4b[...])
    d2v = relu(_mm_rt(x, h5w[...]) + h5b[...])
    w0_out[...] = _mm_rt(relu(_mm_rt(d2v, wa1w[...]) + wa1b[...]),
                         wa2w[...]) + wa2b[...]
    w1_out[...] = _mm_rt(relu(_mm_rt(d2v, wb1w[...]) + wb1b[...]),
                         wb2w[...]) + wb2b[...]


def _softmax_rows(s):
    m = jnp.max(s, axis=-1, keepdims=True)
    e = jnp.exp(s - m)
    return e / (jnp.sum(e, axis=-1, keepdims=True) + 1e-16)


def _attend(h, a_st, a_d, out, masked):
    s_parts = []
    for u in range(_GU):
        for g in range(2):
            row = a_st[g][0:1, u * _CN:(u + 1) * _CN]
            col = a_d[g][u * _CN:(u + 1) * _CN, 0:1]
            s_parts.append(row + col)
    coef = _softmax_rows(masked(jnp.concatenate(s_parts, axis=0)))
    x_parts = []
    for u in range(_GU):
        aggs = [_mm(coef[(2 * u + g) * _CN:(2 * u + g + 1) * _CN, :],
                    h[u * _CN:(u + 1) * _CN, g * out:(g + 1) * out])
                for g in range(2)]
        x_parts.append(jnp.concatenate(aggs, axis=-1))
    return jnp.concatenate(x_parts, axis=0)


def _gnn_body(xcol_ref, w0c0_ref, w0_ref, w1_ref, lw1_ref,
              wo_ref, bo_ref, out_ref):
    nw = _GU * 2 * _CN
    ii = (lax.broadcasted_iota(jnp.int32, (nw, _CN), 0) % _CN) // _NX
    jj = lax.broadcasted_iota(jnp.int32, (nw, _CN), 1) // _NX
    mask = ii == jj

    def masked(s):
        s = jnp.where(s >= 0.0, s, 0.2 * s)
        return jnp.where(mask, s, _NEG)

    w0 = w0_ref[0]
    w1 = w1_ref[0]
    xcol = xcol_ref[0]
    h = xcol * w0c0_ref[0]
    a_st, a_d = [], []
    for g in range(2):
        hg = h[:, 32 * g:32 * (g + 1)]
        a_st.append(_mm_rt(w0[0:1, 128 + 32 * g:160 + 32 * g], hg))
        a_d.append(_mm_rt(hg, w0[0:1, 192 + 32 * g:224 + 32 * g]))
    x2 = _attend(h, a_st, a_d, 32, masked) + w0[0:1, 256:320]

    h2 = _mm_rt(x2, lw1_ref[0])
    a_st2, a_d2 = [], []
    for g in range(2):
        h2g = h2[:, 8 * g:8 * (g + 1)]
        a_st2.append(_mm_rt(w1[0:1, 1024 + 8 * g:1032 + 8 * g], h2g))
        a_d2.append(_mm_rt(h2g, w1[0:1, 1040 + 8 * g:1048 + 8 * g]))
    x3 = _attend(h2, a_st2, a_d2, 8, masked) + w1[0:1, 1056:1072]

    row = jnp.sum(x3.reshape(_NR, _NX, 16), axis=1)
    out_ref[0] = _mm_rt(row, wo_ref[...]) + bo_ref[...]


def _full(shape):
    nd = len(shape)
    return pl.BlockSpec(shape, lambda b: (0,) * nd)


def _per_batch(shape):
    nd = len(shape)
    return pl.BlockSpec((1,) + shape, lambda b: (b,) + (0,) * nd)


@jax.jit
def kernel(xs, pairs, params):
    p = params

    def wt(name):
        W, b = p[name]
        return W.astype(_F32), b.reshape(1, -1).astype(_F32)

    names = ["f1", "f2r", "f3r", "f4r", "f5", "g1", "g2",
             "h1", "h2r", "h3r", "h4r", "h5",
             "wg0_1", "wg0_2", "wg1_1", "wg1_2"]
    wargs = []
    for n in names:
        W, b = wt(n)
        wargs += [W, b]

    pairs_f = pairs.reshape(_BS * _NN, 2)
    w0, w1 = pl.pallas_call(
        _d2v_body,
        out_shape=[jax.ShapeDtypeStruct((_BS, 320), _F32),
                   jax.ShapeDtypeStruct((_BS, 1072), _F32)],
    )(pairs_f, *wargs)

    w0c0 = w0[:, 0:128:2].reshape(_BS, 1, 64)
    lin_w1 = w1[:, :1024].reshape(_BS, 16, 64)

    Wo, bo = p["out_lin"]
    Wo = Wo.astype(_F32)
    bo = bo.reshape(1, 2).astype(_F32)

    xcol = xs.reshape(_BS, _NN, 1).astype(_F32)
    w0r = w0.reshape(_BS, 1, 320)
    w1r = w1.reshape(_BS, 1, 1072)

    out = pl.pallas_call(
        _gnn_body,
        grid=(_BS,),
        in_specs=[_per_batch((_NN, 1)), _per_batch((1, 64)),
                  _per_batch((1, 320)), _per_batch((1, 1072)),
                  _per_batch((16, 64)),
                  _full((2, 16)), _full((1, 2))],
        out_specs=_per_batch((_NR, 2)),
        out_shape=jax.ShapeDtypeStruct((_BS, _NR, 2), _F32),
    )(xcol, w0c0, w0r, w1r, lin_w1, Wo, bo)
    return out
